# Initial kernel scaffold; baseline (speedup 1.0000x reference)
import jax
import jax.numpy as jnp
from jax.experimental import pallas as pl


def kernel(features, targets, unlabeled_features, protos):
    raise NotImplementedError("write your pallas kernel here")


# speedup vs baseline: 2.1083x; 2.1083x over previous
import jax
import jax.numpy as jnp
from jax.experimental import pallas as pl

B = 4096
KP = 5000
NP = 5120
D = 512
C = 50
EPS = 0.05
TEMP = 0.1
MB = 512
NB = 512
NM = B // MB
NN = NP // NB

_f32 = jnp.float32


def _dot(a, b, dims):
    return jax.lax.dot_general(a, b, (dims, ((), ())),
                               preferred_element_type=_f32)


def _mm_exp_kernel(f_ref, p_ref, e_ref, cs_ref):
    n, m = pl.program_id(0), pl.program_id(1)
    out = _dot(f_ref[...], p_ref[...], ((1,), (1,)))
    col = n * NB + jax.lax.broadcasted_iota(jnp.int32, (MB, NB), 1)
    e = jnp.where(col < KP, jnp.exp(out * (1.0 / EPS)), 0.0)
    e_ref[...] = e
    cs_ref[pl.ds(m, 1), :] = jnp.sum(e, axis=0, keepdims=True)


def _mm_exp(f, p):
    outs = [
        jax.ShapeDtypeStruct((B, NP), _f32),
        jax.ShapeDtypeStruct((NM, NP), _f32),
    ]
    specs = [
        pl.BlockSpec((MB, NB), lambda n, m: (m, n)),
        pl.BlockSpec((NM, NB), lambda n, m: (0, n)),
    ]
    return pl.pallas_call(
        _mm_exp_kernel,
        grid=(NN, NM),
        in_specs=[
            pl.BlockSpec((MB, D), lambda n, m: (m, 0)),
            pl.BlockSpec((NB, D), lambda n, m: (n, 0)),
        ],
        out_specs=specs,
        out_shape=outs,
    )(f, p)


def _colsum_kernel(e_ref, v_ref, cs_ref):
    m = pl.program_id(1)
    val = _dot(v_ref[...], e_ref[...], ((0,), (0,)))
    cs_ref[pl.ds(m, 1), :] = val


def _colsum_weighted(e, v):
    csp = pl.pallas_call(
        _colsum_kernel,
        grid=(NN, NM),
        in_specs=[
            pl.BlockSpec((MB, NB), lambda n, m: (m, n)),
            pl.BlockSpec((MB, 1), lambda n, m: (m, 0)),
        ],
        out_specs=pl.BlockSpec((NM, NB), lambda n, m: (0, n)),
        out_shape=jax.ShapeDtypeStruct((NM, NP), _f32),
    )(e, v)
    return jnp.sum(csp, axis=0, keepdims=True)


def _rowsum_kernel(e_ref, u_ref, rs_ref):
    n = pl.program_id(1)

    @pl.when(n == 0)
    def _():
        rs_ref[...] = jnp.zeros_like(rs_ref)
    rs_ref[...] += _dot(e_ref[...], u_ref[...], ((1,), (1,)))


def _rowsum_weighted(e, u):
    return pl.pallas_call(
        _rowsum_kernel,
        grid=(NM, NN),
        in_specs=[
            pl.BlockSpec((MB, NB), lambda m, n: (m, n)),
            pl.BlockSpec((1, NB), lambda m, n: (0, n)),
        ],
        out_specs=pl.BlockSpec((MB, 1), lambda m, n: (m, 0)),
        out_shape=jax.ShapeDtypeStruct((B, 1), _f32),
    )(e, u)


def _sinkhorn_u(e, cs0):
    def u_of(cs):
        return 1.0 / (KP * jnp.maximum(cs, 1e-30))
    u = u_of(cs0)
    for _ in range(2):
        v = 1.0 / (B * jnp.maximum(_rowsum_weighted(e, u), 1e-30))
        u = u_of(_colsum_weighted(e, v))
    kmask = (jnp.arange(NP)[None, :] < KP)
    return jnp.where(kmask, u, 0.0)


MSEL = 128
NSEL = B // MSEL


def _select_kernel(e_ref, u_ref, t_ref, sel_ref, cs_ref, s_ref):
    m = pl.program_id(0)
    e = e_ref[...]
    u = u_ref[...]
    tgt = t_ref[...]
    col = jax.lax.broadcasted_iota(jnp.int32, (MSEL, NP), 1)
    cand = (jnp.remainder(col, C) == tgt) & (col < KP)
    w = jnp.where(cand, e * u, 0.0)
    ww = w
    sel = jnp.zeros_like(w)
    for _ in range(5):
        mx = jnp.max(ww, axis=1, keepdims=True)
        first = jnp.min(jnp.where(ww == mx, col, NP + 1), axis=1,
                        keepdims=True)
        hit = (col == first).astype(_f32)
        sel = sel + hit
        ww = jnp.where(hit > 0.0, -1.0, ww)
    wsel = sel * w
    s = jnp.sum(wsel, axis=1, keepdims=True)
    s_ref[...] = s
    sel_ref[...] = sel
    wn = wsel / jnp.maximum(s, 1e-12)
    cs_ref[pl.ds(m, 1), :] = jnp.sum(wn, axis=0, keepdims=True)


def _select5(e, u, targets2d):
    return pl.pallas_call(
        _select_kernel,
        grid=(NSEL,),
        in_specs=[
            pl.BlockSpec((MSEL, NP), lambda m: (m, 0)),
            pl.BlockSpec((1, NP), lambda m: (0, 0)),
            pl.BlockSpec((MSEL, 1), lambda m: (m, 0)),
        ],
        out_specs=[
            pl.BlockSpec((MSEL, NP), lambda m: (m, 0)),
            pl.BlockSpec((NSEL, NP), lambda m: (0, 0)),
            pl.BlockSpec((MSEL, 1), lambda m: (m, 0)),
        ],
        out_shape=[
            jax.ShapeDtypeStruct((B, NP), _f32),
            jax.ShapeDtypeStruct((NSEL, NP), _f32),
            jax.ShapeDtypeStruct((B, 1), _f32),
        ],
    )(e, u, targets2d)


def _update_kernel(sel_ref, e_ref, u_ref, fs_ref, p_ref, cs_ref, pn_ref):
    m = pl.program_id(1)
    pw = sel_ref[...] * e_ref[...] * u_ref[...]
    contrib = _dot(pw, fs_ref[...], ((0,), (0,)))

    @pl.when(m == 0)
    def _():
        pn_ref[...] = jnp.zeros_like(pn_ref)
    pn_ref[...] += contrib

    @pl.when(m == NM - 1)
    def _():
        uf = pn_ref[...] / jnp.maximum(cs_ref[...], 1e-12)
        pn = 0.99 * p_ref[...] + 0.01 * uf
        nrm = jnp.sqrt(jnp.sum(pn * pn, axis=1, keepdims=True))
        pn_ref[...] = pn / jnp.maximum(nrm, 1e-12)


def _update_protos(sel, e, u, f_scaled, protos_pad, cs_col):
    return pl.pallas_call(
        _update_kernel,
        grid=(NN, NM),
        in_specs=[
            pl.BlockSpec((MB, NB), lambda n, m: (m, n)),
            pl.BlockSpec((MB, NB), lambda n, m: (m, n)),
            pl.BlockSpec((1, NB), lambda n, m: (0, n)),
            pl.BlockSpec((MB, D), lambda n, m: (m, 0)),
            pl.BlockSpec((NB, D), lambda n, m: (n, 0)),
            pl.BlockSpec((NB, 1), lambda n, m: (n, 0)),
        ],
        out_specs=pl.BlockSpec((NB, D), lambda n, m: (n, 0)),
        out_shape=jax.ShapeDtypeStruct((NP, D), _f32),
    )(sel, e, u, f_scaled, protos_pad, cs_col)


def _pos_kernel(sel_ref, e_ref, u_ref, num_ref, den_ref, lse_ref):
    n = pl.program_id(1)
    sel = sel_ref[...]
    e = e_ref[...]
    q = sel * e * u_ref[...]
    logit = jnp.where(sel > 0.0,
                      (EPS / TEMP) * jnp.log(jnp.maximum(e, 1e-38)), 0.0)
    term = q * logit

    @pl.when(n == 0)
    def _():
        num_ref[...] = jnp.zeros_like(num_ref)
        den_ref[...] = jnp.zeros_like(den_ref)
        lse_ref[...] = jnp.zeros_like(lse_ref)
    num_ref[...] += jnp.sum(term, axis=1, keepdims=True)
    den_ref[...] += jnp.sum(q, axis=1, keepdims=True)
    lse_ref[...] += jnp.sum(jnp.sqrt(e), axis=1, keepdims=True)


def _pos_sums(sel, e2, u2):
    return pl.pallas_call(
        _pos_kernel,
        grid=(NM, NN),
        in_specs=[
            pl.BlockSpec((MB, NB), lambda m, n: (m, n)),
            pl.BlockSpec((MB, NB), lambda m, n: (m, n)),
            pl.BlockSpec((1, NB), lambda m, n: (0, n)),
        ],
        out_specs=[
            pl.BlockSpec((MB, 1), lambda m, n: (m, 0)),
            pl.BlockSpec((MB, 1), lambda m, n: (m, 0)),
            pl.BlockSpec((MB, 1), lambda m, n: (m, 0)),
        ],
        out_shape=[
            jax.ShapeDtypeStruct((B, 1), _f32),
            jax.ShapeDtypeStruct((B, 1), _f32),
            jax.ShapeDtypeStruct((B, 1), _f32),
        ],
    )(sel, e2, u2)


def _pcon_kernel(pi_ref, pj_ref, ci_ref, cj_ref, pos_ref, neg_ref):
    i, j = pl.program_id(0), pl.program_id(1)
    g = _dot(pi_ref[...], pj_ref[...], ((1,), (1,)))
    x = 2.0 * g
    gi = i * NB + jax.lax.broadcasted_iota(jnp.int32, (NB, NB), 0)
    gj = j * NB + jax.lax.broadcasted_iota(jnp.int32, (NB, NB), 1)
    offd = (gi != gj) & (gj < KP)
    samec = offd & (ci_ref[...] == cj_ref[...])

    @pl.when(j == 0)
    def _():
        pos_ref[...] = jnp.zeros_like(pos_ref)
        neg_ref[...] = jnp.zeros_like(neg_ref)
    pos_ref[...] += jnp.sum(jnp.where(samec, x, 0.0), axis=1, keepdims=True)
    neg_ref[...] += jnp.sum(jnp.where(offd, jnp.exp(x - 2.0), 0.0),
                            axis=1, keepdims=True)


def _pcon_sums(pn, cls_col, cls_row):
    return pl.pallas_call(
        _pcon_kernel,
        grid=(NN, NN),
        in_specs=[
            pl.BlockSpec((NB, D), lambda i, j: (i, 0)),
            pl.BlockSpec((NB, D), lambda i, j: (j, 0)),
            pl.BlockSpec((NB, 1), lambda i, j: (i, 0)),
            pl.BlockSpec((1, NB), lambda i, j: (0, j)),
        ],
        out_specs=[
            pl.BlockSpec((NB, 1), lambda i, j: (i, 0)),
            pl.BlockSpec((NB, 1), lambda i, j: (i, 0)),
        ],
        out_shape=[
            jax.ShapeDtypeStruct((NP, 1), _f32),
            jax.ShapeDtypeStruct((NP, 1), _f32),
        ],
    )(pn, pn, cls_col, cls_row)


def _ulmax_kernel(x_ref, p_ref, mx_ref):
    n = pl.program_id(1)
    sims = _dot(x_ref[...], p_ref[...], ((1,), (1,)))
    col = n * NB + jax.lax.broadcasted_iota(jnp.int32, (MB, NB), 1)
    sims = jnp.where(col < KP, sims, -3e38)
    blockmax = jnp.max(sims, axis=1, keepdims=True)

    @pl.when(n == 0)
    def _():
        mx_ref[...] = jnp.full_like(mx_ref, -3e38)
    mx_ref[...] = jnp.maximum(mx_ref[...], blockmax)


def _ul_max(x, pn):
    return pl.pallas_call(
        _ulmax_kernel,
        grid=(NM, NN),
        in_specs=[
            pl.BlockSpec((MB, D), lambda m, n: (m, 0)),
            pl.BlockSpec((NB, D), lambda m, n: (n, 0)),
        ],
        out_specs=pl.BlockSpec((MB, 1), lambda m, n: (m, 0)),
        out_shape=jax.ShapeDtypeStruct((B, 1), _f32),
    )(x, pn)


def kernel(features, targets, unlabeled_features, protos):
    f = features.astype(_f32)
    protos_pad = jnp.pad(protos.astype(_f32), ((0, NP - KP), (0, 0)))
    targets2d = targets.astype(jnp.int32).reshape(B, 1)

    e1, csp1 = _mm_exp(f, protos_pad)
    u1 = _sinkhorn_u(e1, jnp.sum(csp1, axis=0, keepdims=True))

    sel, csp_w, s = _select5(e1, u1, targets2d)
    cs_col = jnp.sum(csp_w, axis=0).reshape(NP, 1)
    f_scaled = f / jnp.maximum(s, 1e-12)
    pn = _update_protos(sel, e1, u1, f_scaled, protos_pad, cs_col)

    e2, csp2 = _mm_exp(f, pn)
    u2 = _sinkhorn_u(e2, jnp.sum(csp2, axis=0, keepdims=True))
    num, den, lse_rs = _pos_sums(sel, e2, u2)
    neg = jnp.log(lse_rs[:, 0])
    pos = num[:, 0] / jnp.maximum(den[:, 0], 1e-30)
    mle = jnp.mean(neg) - jnp.mean(pos)

    cls = jnp.remainder(jnp.arange(NP, dtype=jnp.int32), C).astype(_f32)
    posa, nega = _pcon_sums(pn, cls.reshape(NP, 1), cls.reshape(1, NP))
    pcon_rows = posa[:KP, 0] / 99.0 - (jnp.log(nega[:KP, 0]) + 2.0)
    pcon = -jnp.mean(pcon_rows)

    mx = _ul_max(unlabeled_features.astype(_f32), pn)
    ul = 0.5 * jnp.mean(1.0 - mx[:, 0])

    return mle + pcon + ul


# speedup vs baseline: 2.2880x; 1.0852x over previous
import jax
import jax.numpy as jnp
from jax.experimental import pallas as pl

B = 4096
KP = 5000
NP = 5120
D = 512
C = 50
EPS = 0.05
TEMP = 0.1
MB = 512
NB = 512
NM = B // MB
NN = NP // NB

_f32 = jnp.float32


def _dot(a, b, dims):
    return jax.lax.dot_general(a, b, (dims, ((), ())),
                               preferred_element_type=_f32)


def _mm_exp_kernel(f_ref, p_ref, e_ref, cs_ref):
    n, m = pl.program_id(0), pl.program_id(1)
    out = _dot(f_ref[...], p_ref[...], ((1,), (1,)))
    col = n * NB + jax.lax.broadcasted_iota(jnp.int32, (MB, NB), 1)
    e = jnp.where(col < KP, jnp.exp(out * (1.0 / EPS)), 0.0)
    e_ref[...] = e.astype(jnp.bfloat16)
    cs_ref[pl.ds(m, 1), :] = jnp.sum(e, axis=0, keepdims=True)


def _mm_exp(f, p):
    outs = [
        jax.ShapeDtypeStruct((B, NP), jnp.bfloat16),
        jax.ShapeDtypeStruct((NM, NP), _f32),
    ]
    specs = [
        pl.BlockSpec((MB, NB), lambda n, m: (m, n)),
        pl.BlockSpec((NM, NB), lambda n, m: (0, n)),
    ]
    return pl.pallas_call(
        _mm_exp_kernel,
        grid=(NN, NM),
        in_specs=[
            pl.BlockSpec((MB, D), lambda n, m: (m, 0)),
            pl.BlockSpec((NB, D), lambda n, m: (n, 0)),
        ],
        out_specs=specs,
        out_shape=outs,
    )(f, p)


def _colsum_kernel(e_ref, v_ref, cs_ref):
    m = pl.program_id(1)
    e = e_ref[...].astype(_f32)
    val = _dot(v_ref[...], e, ((0,), (0,)))
    cs_ref[pl.ds(m, 1), :] = val


def _colsum_weighted(e, v):
    csp = pl.pallas_call(
        _colsum_kernel,
        grid=(NN, NM),
        in_specs=[
            pl.BlockSpec((MB, NB), lambda n, m: (m, n)),
            pl.BlockSpec((MB, 1), lambda n, m: (m, 0)),
        ],
        out_specs=pl.BlockSpec((NM, NB), lambda n, m: (0, n)),
        out_shape=jax.ShapeDtypeStruct((NM, NP), _f32),
    )(e, v)
    return jnp.sum(csp, axis=0, keepdims=True)


def _rowsum_kernel(e_ref, u_ref, rs_ref):
    n = pl.program_id(1)

    @pl.when(n == 0)
    def _():
        rs_ref[...] = jnp.zeros_like(rs_ref)
    rs_ref[...] += _dot(e_ref[...].astype(_f32), u_ref[...],
                        ((1,), (1,)))


def _rowsum_weighted(e, u):
    return pl.pallas_call(
        _rowsum_kernel,
        grid=(NM, NN),
        in_specs=[
            pl.BlockSpec((MB, NB), lambda m, n: (m, n)),
            pl.BlockSpec((1, NB), lambda m, n: (0, n)),
        ],
        out_specs=pl.BlockSpec((MB, 1), lambda m, n: (m, 0)),
        out_shape=jax.ShapeDtypeStruct((B, 1), _f32),
    )(e, u)


def _sinkhorn_u(e, cs0):
    def u_of(cs):
        return 1.0 / (KP * jnp.maximum(cs, 1e-30))
    u = u_of(cs0)
    for _ in range(2):
        v = 1.0 / (B * jnp.maximum(_rowsum_weighted(e, u), 1e-30))
        u = u_of(_colsum_weighted(e, v))
    kmask = (jnp.arange(NP)[None, :] < KP)
    return jnp.where(kmask, u, 0.0)


MSEL = 128
NSEL = B // MSEL


def _select_kernel(e_ref, u_ref, t_ref, sel_ref, cs_ref, s_ref):
    m = pl.program_id(0)
    e = e_ref[...].astype(_f32)
    u = u_ref[...]
    tgt = t_ref[...]
    col = jax.lax.broadcasted_iota(jnp.int32, (MSEL, NP), 1)
    cand = (jnp.remainder(col, C) == tgt) & (col < KP)
    w = jnp.where(cand, e * u, 0.0)
    ww = w
    sel = jnp.zeros_like(w)
    for _ in range(5):
        mx = jnp.max(ww, axis=1, keepdims=True)
        first = jnp.min(jnp.where(ww == mx, col, NP + 1), axis=1,
                        keepdims=True)
        hit = (col == first).astype(_f32)
        sel = sel + hit
        ww = jnp.where(hit > 0.0, -1.0, ww)
    wsel = sel * w
    s = jnp.sum(wsel, axis=1, keepdims=True)
    s_ref[...] = s
    sel_ref[...] = sel.astype(jnp.bfloat16)
    wn = wsel / jnp.maximum(s, 1e-12)
    cs_ref[pl.ds(m, 1), :] = jnp.sum(wn, axis=0, keepdims=True)


def _select5(e, u, targets2d):
    return pl.pallas_call(
        _select_kernel,
        grid=(NSEL,),
        in_specs=[
            pl.BlockSpec((MSEL, NP), lambda m: (m, 0)),
            pl.BlockSpec((1, NP), lambda m: (0, 0)),
            pl.BlockSpec((MSEL, 1), lambda m: (m, 0)),
        ],
        out_specs=[
            pl.BlockSpec((MSEL, NP), lambda m: (m, 0)),
            pl.BlockSpec((NSEL, NP), lambda m: (0, 0)),
            pl.BlockSpec((MSEL, 1), lambda m: (m, 0)),
        ],
        out_shape=[
            jax.ShapeDtypeStruct((B, NP), jnp.bfloat16),
            jax.ShapeDtypeStruct((NSEL, NP), _f32),
            jax.ShapeDtypeStruct((B, 1), _f32),
        ],
    )(e, u, targets2d)


def _update_kernel(sel_ref, e_ref, u_ref, fs_ref, p_ref, cs_ref, pn_ref):
    m = pl.program_id(1)
    pw = (sel_ref[...].astype(_f32) * e_ref[...].astype(_f32)
          * u_ref[...])
    contrib = _dot(pw, fs_ref[...], ((0,), (0,)))

    @pl.when(m == 0)
    def _():
        pn_ref[...] = jnp.zeros_like(pn_ref)
    pn_ref[...] += contrib

    @pl.when(m == NM - 1)
    def _():
        uf = pn_ref[...] / jnp.maximum(cs_ref[...], 1e-12)
        pn = 0.99 * p_ref[...] + 0.01 * uf
        nrm = jnp.sqrt(jnp.sum(pn * pn, axis=1, keepdims=True))
        pn_ref[...] = pn / jnp.maximum(nrm, 1e-12)


def _update_protos(sel, e, u, f_scaled, protos_pad, cs_col):
    return pl.pallas_call(
        _update_kernel,
        grid=(NN, NM),
        in_specs=[
            pl.BlockSpec((MB, NB), lambda n, m: (m, n)),
            pl.BlockSpec((MB, NB), lambda n, m: (m, n)),
            pl.BlockSpec((1, NB), lambda n, m: (0, n)),
            pl.BlockSpec((MB, D), lambda n, m: (m, 0)),
            pl.BlockSpec((NB, D), lambda n, m: (n, 0)),
            pl.BlockSpec((NB, 1), lambda n, m: (n, 0)),
        ],
        out_specs=pl.BlockSpec((NB, D), lambda n, m: (n, 0)),
        out_shape=jax.ShapeDtypeStruct((NP, D), _f32),
    )(sel, e, u, f_scaled, protos_pad, cs_col)


def _pos_kernel(sel_ref, e_ref, u_ref, num_ref, den_ref, lse_ref):
    n = pl.program_id(1)
    sel = sel_ref[...].astype(_f32)
    e = e_ref[...].astype(_f32)
    q = sel * e * u_ref[...]
    logit = jnp.where(sel > 0.0,
                      (EPS / TEMP) * jnp.log(jnp.maximum(e, 1e-38)), 0.0)
    term = q * logit

    @pl.when(n == 0)
    def _():
        num_ref[...] = jnp.zeros_like(num_ref)
        den_ref[...] = jnp.zeros_like(den_ref)
        lse_ref[...] = jnp.zeros_like(lse_ref)
    num_ref[...] += jnp.sum(term, axis=1, keepdims=True)
    den_ref[...] += jnp.sum(q, axis=1, keepdims=True)
    lse_ref[...] += jnp.sum(jnp.sqrt(e), axis=1, keepdims=True)


def _pos_sums(sel, e2, u2):
    return pl.pallas_call(
        _pos_kernel,
        grid=(NM, NN),
        in_specs=[
            pl.BlockSpec((MB, NB), lambda m, n: (m, n)),
            pl.BlockSpec((MB, NB), lambda m, n: (m, n)),
            pl.BlockSpec((1, NB), lambda m, n: (0, n)),
        ],
        out_specs=[
            pl.BlockSpec((MB, 1), lambda m, n: (m, 0)),
            pl.BlockSpec((MB, 1), lambda m, n: (m, 0)),
            pl.BlockSpec((MB, 1), lambda m, n: (m, 0)),
        ],
        out_shape=[
            jax.ShapeDtypeStruct((B, 1), _f32),
            jax.ShapeDtypeStruct((B, 1), _f32),
            jax.ShapeDtypeStruct((B, 1), _f32),
        ],
    )(sel, e2, u2)


def _pcon_kernel(pi_ref, pj_ref, ci_ref, cj_ref, pos_ref, neg_ref):
    i, j = pl.program_id(0), pl.program_id(1)
    g = _dot(pi_ref[...], pj_ref[...], ((1,), (1,)))
    x = 2.0 * g
    gi = i * NB + jax.lax.broadcasted_iota(jnp.int32, (NB, NB), 0)
    gj = j * NB + jax.lax.broadcasted_iota(jnp.int32, (NB, NB), 1)
    offd = (gi != gj) & (gj < KP)
    samec = offd & (ci_ref[...] == cj_ref[...])

    @pl.when(j == 0)
    def _():
        pos_ref[...] = jnp.zeros_like(pos_ref)
        neg_ref[...] = jnp.zeros_like(neg_ref)
    pos_ref[...] += jnp.sum(jnp.where(samec, x, 0.0), axis=1, keepdims=True)
    neg_ref[...] += jnp.sum(jnp.where(offd, jnp.exp(x - 2.0), 0.0),
                            axis=1, keepdims=True)


def _pcon_sums(pn, cls_col, cls_row):
    return pl.pallas_call(
        _pcon_kernel,
        grid=(NN, NN),
        in_specs=[
            pl.BlockSpec((NB, D), lambda i, j: (i, 0)),
            pl.BlockSpec((NB, D), lambda i, j: (j, 0)),
            pl.BlockSpec((NB, 1), lambda i, j: (i, 0)),
            pl.BlockSpec((1, NB), lambda i, j: (0, j)),
        ],
        out_specs=[
            pl.BlockSpec((NB, 1), lambda i, j: (i, 0)),
            pl.BlockSpec((NB, 1), lambda i, j: (i, 0)),
        ],
        out_shape=[
            jax.ShapeDtypeStruct((NP, 1), _f32),
            jax.ShapeDtypeStruct((NP, 1), _f32),
        ],
    )(pn, pn, cls_col, cls_row)


def _ulmax_kernel(x_ref, p_ref, mx_ref):
    n = pl.program_id(1)
    sims = _dot(x_ref[...], p_ref[...], ((1,), (1,)))
    col = n * NB + jax.lax.broadcasted_iota(jnp.int32, (MB, NB), 1)
    sims = jnp.where(col < KP, sims, -3e38)
    blockmax = jnp.max(sims, axis=1, keepdims=True)

    @pl.when(n == 0)
    def _():
        mx_ref[...] = jnp.full_like(mx_ref, -3e38)
    mx_ref[...] = jnp.maximum(mx_ref[...], blockmax)


def _ul_max(x, pn):
    return pl.pallas_call(
        _ulmax_kernel,
        grid=(NM, NN),
        in_specs=[
            pl.BlockSpec((MB, D), lambda m, n: (m, 0)),
            pl.BlockSpec((NB, D), lambda m, n: (n, 0)),
        ],
        out_specs=pl.BlockSpec((MB, 1), lambda m, n: (m, 0)),
        out_shape=jax.ShapeDtypeStruct((B, 1), _f32),
    )(x, pn)


def kernel(features, targets, unlabeled_features, protos):
    f = features.astype(_f32)
    protos_pad = jnp.pad(protos.astype(_f32), ((0, NP - KP), (0, 0)))
    targets2d = targets.astype(jnp.int32).reshape(B, 1)

    e1, csp1 = _mm_exp(f, protos_pad)
    u1 = _sinkhorn_u(e1, jnp.sum(csp1, axis=0, keepdims=True))

    sel, csp_w, s = _select5(e1, u1, targets2d)
    cs_col = jnp.sum(csp_w, axis=0).reshape(NP, 1)
    f_scaled = f / jnp.maximum(s, 1e-12)
    pn = _update_protos(sel, e1, u1, f_scaled, protos_pad, cs_col)

    e2, csp2 = _mm_exp(f, pn)
    u2 = _sinkhorn_u(e2, jnp.sum(csp2, axis=0, keepdims=True))
    num, den, lse_rs = _pos_sums(sel, e2, u2)
    neg = jnp.log(lse_rs[:, 0])
    pos = num[:, 0] / jnp.maximum(den[:, 0], 1e-30)
    mle = jnp.mean(neg) - jnp.mean(pos)

    cls = jnp.remainder(jnp.arange(NP, dtype=jnp.int32), C).astype(_f32)
    posa, nega = _pcon_sums(pn, cls.reshape(NP, 1), cls.reshape(1, NP))
    pcon_rows = posa[:KP, 0] / 99.0 - (jnp.log(nega[:KP, 0]) + 2.0)
    pcon = -jnp.mean(pcon_rows)

    mx = _ul_max(unlabeled_features.astype(_f32), pn)
    ul = 0.5 * jnp.mean(1.0 - mx[:, 0])

    return mle + pcon + ul


# speedup vs baseline: 2.3557x; 1.0296x over previous
import functools

import jax
import jax.numpy as jnp
from jax import lax
from jax.experimental import pallas as pl
from jax.experimental.pallas import tpu as pltpu
from jax.experimental.pallas import tpu_sc as plsc

B = 4096
KP = 5000
NP = 5120
D = 512
C = 50
EPS = 0.05
TEMP = 0.1
MB = 512
NB = 512
NM = B // MB
NN = NP // NB

_f32 = jnp.float32


def _dot(a, b, dims):
    return jax.lax.dot_general(a, b, (dims, ((), ())),
                               preferred_element_type=_f32)


def _mm_exp_kernel(f_ref, p_ref, e_ref, cs_ref):
    n, m = pl.program_id(0), pl.program_id(1)
    out = _dot(f_ref[...], p_ref[...], ((1,), (1,)))
    col = n * NB + jax.lax.broadcasted_iota(jnp.int32, (MB, NB), 1)
    e = jnp.where(col < KP, jnp.exp(out * (1.0 / EPS)), 0.0)
    e_ref[...] = e.astype(jnp.bfloat16)
    cs_ref[pl.ds(m, 1), :] = jnp.sum(e, axis=0, keepdims=True)


def _mm_exp(f, p):
    outs = [
        jax.ShapeDtypeStruct((B, NP), jnp.bfloat16),
        jax.ShapeDtypeStruct((NM, NP), _f32),
    ]
    specs = [
        pl.BlockSpec((MB, NB), lambda n, m: (m, n)),
        pl.BlockSpec((NM, NB), lambda n, m: (0, n)),
    ]
    return pl.pallas_call(
        _mm_exp_kernel,
        grid=(NN, NM),
        in_specs=[
            pl.BlockSpec((MB, D), lambda n, m: (m, 0)),
            pl.BlockSpec((NB, D), lambda n, m: (n, 0)),
        ],
        out_specs=specs,
        out_shape=outs,
    )(f, p)


def _colsum_kernel(e_ref, v_ref, cs_ref):
    m = pl.program_id(1)
    e = e_ref[...].astype(_f32)
    val = _dot(v_ref[...], e, ((0,), (0,)))
    cs_ref[pl.ds(m, 1), :] = val


def _colsum_weighted(e, v):
    csp = pl.pallas_call(
        _colsum_kernel,
        grid=(NN, NM),
        in_specs=[
            pl.BlockSpec((MB, NB), lambda n, m: (m, n)),
            pl.BlockSpec((MB, 1), lambda n, m: (m, 0)),
        ],
        out_specs=pl.BlockSpec((NM, NB), lambda n, m: (0, n)),
        out_shape=jax.ShapeDtypeStruct((NM, NP), _f32),
    )(e, v)
    return jnp.sum(csp, axis=0, keepdims=True)


def _rowsum_kernel(e_ref, u_ref, rs_ref):
    n = pl.program_id(1)

    @pl.when(n == 0)
    def _():
        rs_ref[...] = jnp.zeros_like(rs_ref)
    rs_ref[...] += _dot(e_ref[...].astype(_f32), u_ref[...],
                        ((1,), (1,)))


def _rowsum_weighted(e, u):
    return pl.pallas_call(
        _rowsum_kernel,
        grid=(NM, NN),
        in_specs=[
            pl.BlockSpec((MB, NB), lambda m, n: (m, n)),
            pl.BlockSpec((1, NB), lambda m, n: (0, n)),
        ],
        out_specs=pl.BlockSpec((MB, 1), lambda m, n: (m, 0)),
        out_shape=jax.ShapeDtypeStruct((B, 1), _f32),
    )(e, u)


def _sinkhorn_u(e, cs0):
    def u_of(cs):
        return 1.0 / (KP * jnp.maximum(cs, 1e-30))
    u = u_of(cs0)
    for _ in range(2):
        v = 1.0 / (B * jnp.maximum(_rowsum_weighted(e, u), 1e-30))
        u = u_of(_colsum_weighted(e, v))
    kmask = (jnp.arange(NP)[None, :] < KP)
    return jnp.where(kmask, u, 0.0)


MSEL = 128
NSEL = B // MSEL
NCAND = KP // C
KC = 128


def _compact_kernel(e_ref, u_ref, t_ref, g_ref, wc_ref):
    e = e_ref[...].astype(_f32)
    tgt = t_ref[...]
    col = jax.lax.broadcasted_iota(jnp.int32, (MB, NP), 1)
    w = jnp.where(jnp.remainder(col, C) == tgt, e * u_ref[...], 0.0)
    wc_ref[...] = _dot(w, g_ref[...], ((1,), (0,)))


def _compact(e, u, targets2d, g):
    return pl.pallas_call(
        _compact_kernel,
        grid=(NM,),
        in_specs=[
            pl.BlockSpec((MB, NP), lambda m: (m, 0)),
            pl.BlockSpec((1, NP), lambda m: (0, 0)),
            pl.BlockSpec((MB, 1), lambda m: (m, 0)),
            pl.BlockSpec((NP, KC), lambda m: (0, 0)),
        ],
        out_specs=pl.BlockSpec((MB, KC), lambda m: (m, 0)),
        out_shape=jax.ShapeDtypeStruct((B, KC), _f32),
    )(e, u, targets2d, g)


_SC_INFO = plsc.get_sparse_core_info()
SCW = _SC_INFO.num_cores * _SC_INFO.num_subcores
RPW = B // SCW


NG = RPW // 16


def _sc_top5(wct_flat):
    mesh = plsc.VectorSubcoreMesh(core_axis_name="c", subcore_axis_name="s")
    nc = _SC_INFO.num_cores

    @functools.partial(
        pl.kernel, mesh=mesh,
        out_type=jax.ShapeDtypeStruct((5 * B,), jnp.int32),
        scratch_types=[
            pltpu.VMEM((RPW * KC,), _f32),
            pltpu.VMEM((5 * RPW,), jnp.int32),
        ],
    )
    def k(wc_hbm, ks_hbm, wv, oi):
        wid = lax.axis_index("s") * nc + lax.axis_index("c")
        base = wid * RPW
        pltpu.sync_copy(wc_hbm.at[pl.ds(base * KC, RPW * KC)], wv)

        def group(g, carry):
            def step(kk, st):
                tv, ti = st
                cv = wv[pl.ds(g * (16 * KC) + kk * 16, 16)]
                ci = kk
                ntv, nti = [], []
                for lvl in range(5):
                    gt = cv > tv[lvl]
                    ntv.append(jnp.where(gt, cv, tv[lvl]))
                    nti.append(jnp.where(gt, ci, ti[lvl]))
                    cv = jnp.where(gt, tv[lvl], cv)
                    ci = jnp.where(gt, ti[lvl], ci)
                return tuple(ntv), tuple(nti)

            init = (tuple(jnp.full((16,), -1.0, _f32) for _ in range(5)),
                    tuple(jnp.zeros((16,), jnp.int32) for _ in range(5)))
            tv, ti = lax.fori_loop(0, KC, step, init)
            for lvl in range(5):
                oi[pl.ds(lvl * RPW + g * 16, 16)] = ti[lvl]
            return carry

        lax.fori_loop(0, NG, group, 0)
        for lvl in range(5):
            pltpu.sync_copy(oi.at[pl.ds(lvl * RPW, RPW)],
                            ks_hbm.at[pl.ds(lvl * B + base, RPW)])

    return k(wct_flat)


def _recon_kernel(e_ref, u_ref, t_ref, k_ref, sel_ref, cs_ref, s_ref):
    m = pl.program_id(0)
    e = e_ref[...].astype(_f32)
    u = u_ref[...]
    tgt = t_ref[...]
    ks = k_ref[...]
    col = jax.lax.broadcasted_iota(jnp.int32, (MSEL, NP), 1)
    sel = jnp.zeros((MSEL, NP), _f32)
    for mm in range(5):
        g = tgt + C * ks[:, mm:mm + 1]
        sel = sel + (col == g).astype(_f32)
    wsel = sel * (e * u)
    s = jnp.sum(wsel, axis=1, keepdims=True)
    s_ref[...] = s
    sel_ref[...] = sel.astype(jnp.bfloat16)
    wn = wsel / jnp.maximum(s, 1e-12)
    cs_ref[pl.ds(m, 1), :] = jnp.sum(wn, axis=0, keepdims=True)


def _select5(e, u, targets2d, ks):
    return pl.pallas_call(
        _recon_kernel,
        grid=(NSEL,),
        in_specs=[
            pl.BlockSpec((MSEL, NP), lambda m: (m, 0)),
            pl.BlockSpec((1, NP), lambda m: (0, 0)),
            pl.BlockSpec((MSEL, 1), lambda m: (m, 0)),
            pl.BlockSpec((MSEL, 16), lambda m: (m, 0)),
        ],
        out_specs=[
            pl.BlockSpec((MSEL, NP), lambda m: (m, 0)),
            pl.BlockSpec((NSEL, NP), lambda m: (0, 0)),
            pl.BlockSpec((MSEL, 1), lambda m: (m, 0)),
        ],
        out_shape=[
            jax.ShapeDtypeStruct((B, NP), jnp.bfloat16),
            jax.ShapeDtypeStruct((NSEL, NP), _f32),
            jax.ShapeDtypeStruct((B, 1), _f32),
        ],
    )(e, u, targets2d, ks)


def _update_kernel(sel_ref, e_ref, u_ref, fs_ref, p_ref, cs_ref, pn_ref):
    m = pl.program_id(1)
    pw = (sel_ref[...].astype(_f32) * e_ref[...].astype(_f32)
          * u_ref[...])
    contrib = _dot(pw, fs_ref[...], ((0,), (0,)))

    @pl.when(m == 0)
    def _():
        pn_ref[...] = jnp.zeros_like(pn_ref)
    pn_ref[...] += contrib

    @pl.when(m == NM - 1)
    def _():
        uf = pn_ref[...] / jnp.maximum(cs_ref[...], 1e-12)
        pn = 0.99 * p_ref[...] + 0.01 * uf
        nrm = jnp.sqrt(jnp.sum(pn * pn, axis=1, keepdims=True))
        pn_ref[...] = pn / jnp.maximum(nrm, 1e-12)


def _update_protos(sel, e, u, f_scaled, protos_pad, cs_col):
    return pl.pallas_call(
        _update_kernel,
        grid=(NN, NM),
        in_specs=[
            pl.BlockSpec((MB, NB), lambda n, m: (m, n)),
            pl.BlockSpec((MB, NB), lambda n, m: (m, n)),
            pl.BlockSpec((1, NB), lambda n, m: (0, n)),
            pl.BlockSpec((MB, D), lambda n, m: (m, 0)),
            pl.BlockSpec((NB, D), lambda n, m: (n, 0)),
            pl.BlockSpec((NB, 1), lambda n, m: (n, 0)),
        ],
        out_specs=pl.BlockSpec((NB, D), lambda n, m: (n, 0)),
        out_shape=jax.ShapeDtypeStruct((NP, D), _f32),
    )(sel, e, u, f_scaled, protos_pad, cs_col)


def _pos_kernel(sel_ref, e_ref, u_ref, num_ref, den_ref, lse_ref):
    n = pl.program_id(1)
    sel = sel_ref[...].astype(_f32)
    e = e_ref[...].astype(_f32)
    q = sel * e * u_ref[...]
    logit = jnp.where(sel > 0.0,
                      (EPS / TEMP) * jnp.log(jnp.maximum(e, 1e-38)), 0.0)
    term = q * logit

    @pl.when(n == 0)
    def _():
        num_ref[...] = jnp.zeros_like(num_ref)
        den_ref[...] = jnp.zeros_like(den_ref)
        lse_ref[...] = jnp.zeros_like(lse_ref)
    num_ref[...] += jnp.sum(term, axis=1, keepdims=True)
    den_ref[...] += jnp.sum(q, axis=1, keepdims=True)
    lse_ref[...] += jnp.sum(jnp.sqrt(e), axis=1, keepdims=True)


def _pos_sums(sel, e2, u2):
    return pl.pallas_call(
        _pos_kernel,
        grid=(NM, NN),
        in_specs=[
            pl.BlockSpec((MB, NB), lambda m, n: (m, n)),
            pl.BlockSpec((MB, NB), lambda m, n: (m, n)),
            pl.BlockSpec((1, NB), lambda m, n: (0, n)),
        ],
        out_specs=[
            pl.BlockSpec((MB, 1), lambda m, n: (m, 0)),
            pl.BlockSpec((MB, 1), lambda m, n: (m, 0)),
            pl.BlockSpec((MB, 1), lambda m, n: (m, 0)),
        ],
        out_shape=[
            jax.ShapeDtypeStruct((B, 1), _f32),
            jax.ShapeDtypeStruct((B, 1), _f32),
            jax.ShapeDtypeStruct((B, 1), _f32),
        ],
    )(sel, e2, u2)


def _pcon_kernel(pi_ref, pj_ref, ci_ref, cj_ref, pos_ref, neg_ref):
    i, j = pl.program_id(0), pl.program_id(1)
    g = _dot(pi_ref[...], pj_ref[...], ((1,), (1,)))
    x = 2.0 * g
    gi = i * NB + jax.lax.broadcasted_iota(jnp.int32, (NB, NB), 0)
    gj = j * NB + jax.lax.broadcasted_iota(jnp.int32, (NB, NB), 1)
    offd = (gi != gj) & (gj < KP)
    samec = offd & (ci_ref[...] == cj_ref[...])

    @pl.when(j == 0)
    def _():
        pos_ref[...] = jnp.zeros_like(pos_ref)
        neg_ref[...] = jnp.zeros_like(neg_ref)
    pos_ref[...] += jnp.sum(jnp.where(samec, x, 0.0), axis=1, keepdims=True)
    neg_ref[...] += jnp.sum(jnp.where(offd, jnp.exp(x - 2.0), 0.0),
                            axis=1, keepdims=True)


def _pcon_sums(pn, cls_col, cls_row):
    return pl.pallas_call(
        _pcon_kernel,
        grid=(NN, NN),
        in_specs=[
            pl.BlockSpec((NB, D), lambda i, j: (i, 0)),
            pl.BlockSpec((NB, D), lambda i, j: (j, 0)),
            pl.BlockSpec((NB, 1), lambda i, j: (i, 0)),
            pl.BlockSpec((1, NB), lambda i, j: (0, j)),
        ],
        out_specs=[
            pl.BlockSpec((NB, 1), lambda i, j: (i, 0)),
            pl.BlockSpec((NB, 1), lambda i, j: (i, 0)),
        ],
        out_shape=[
            jax.ShapeDtypeStruct((NP, 1), _f32),
            jax.ShapeDtypeStruct((NP, 1), _f32),
        ],
    )(pn, pn, cls_col, cls_row)


def _ulmax_kernel(x_ref, p_ref, mx_ref):
    n = pl.program_id(1)
    sims = _dot(x_ref[...], p_ref[...], ((1,), (1,)))
    col = n * NB + jax.lax.broadcasted_iota(jnp.int32, (MB, NB), 1)
    sims = jnp.where(col < KP, sims, -3e38)
    blockmax = jnp.max(sims, axis=1, keepdims=True)

    @pl.when(n == 0)
    def _():
        mx_ref[...] = jnp.full_like(mx_ref, -3e38)
    mx_ref[...] = jnp.maximum(mx_ref[...], blockmax)


def _ul_max(x, pn):
    return pl.pallas_call(
        _ulmax_kernel,
        grid=(NM, NN),
        in_specs=[
            pl.BlockSpec((MB, D), lambda m, n: (m, 0)),
            pl.BlockSpec((NB, D), lambda m, n: (n, 0)),
        ],
        out_specs=pl.BlockSpec((MB, 1), lambda m, n: (m, 0)),
        out_shape=jax.ShapeDtypeStruct((B, 1), _f32),
    )(x, pn)


def kernel(features, targets, unlabeled_features, protos):
    f = features.astype(_f32)
    protos_pad = jnp.pad(protos.astype(_f32), ((0, NP - KP), (0, 0)))
    targets2d = targets.astype(jnp.int32).reshape(B, 1)

    e1, csp1 = _mm_exp(f, protos_pad)
    u1 = _sinkhorn_u(e1, jnp.sum(csp1, axis=0, keepdims=True))

    grp = (jnp.arange(NP)[:, None] // C ==
           jnp.arange(KC)[None, :]).astype(_f32)
    grp = grp * (jnp.arange(NP)[:, None] < KP)
    wc = _compact(e1, u1, targets2d, grp)
    wct = wc.reshape(B // 16, 16, KC).swapaxes(1, 2).reshape(-1)
    ks5 = _sc_top5(wct).reshape(5, B)
    ks = jnp.pad(ks5.T, ((0, 0), (0, 11)))
    sel, csp_w, s = _select5(e1, u1, targets2d, ks)
    cs_col = jnp.sum(csp_w, axis=0).reshape(NP, 1)
    f_scaled = f / jnp.maximum(s, 1e-12)
    pn = _update_protos(sel, e1, u1, f_scaled, protos_pad, cs_col)

    e2, csp2 = _mm_exp(f, pn)
    u2 = _sinkhorn_u(e2, jnp.sum(csp2, axis=0, keepdims=True))
    num, den, lse_rs = _pos_sums(sel, e2, u2)
    neg = jnp.log(lse_rs[:, 0])
    pos = num[:, 0] / jnp.maximum(den[:, 0], 1e-30)
    mle = jnp.mean(neg) - jnp.mean(pos)

    cls = jnp.remainder(jnp.arange(NP, dtype=jnp.int32), C).astype(_f32)
    posa, nega = _pcon_sums(pn, cls.reshape(NP, 1), cls.reshape(1, NP))
    pcon_rows = posa[:KP, 0] / 99.0 - (jnp.log(nega[:KP, 0]) + 2.0)
    pcon = -jnp.mean(pcon_rows)

    mx = _ul_max(unlabeled_features.astype(_f32), pn)
    ul = 0.5 * jnp.mean(1.0 - mx[:, 0])

    return mle + pcon + ul
